# Initial kernel scaffold; baseline (speedup 1.0000x reference)
import jax
import jax.numpy as jnp
from jax.experimental import pallas as pl


def kernel(node_attrs, positions, magmoms, edge_index, W_embed, W_r1, W_m1, W_r2, W_m2, w_read, atomic_energies):
    raise NotImplementedError("write your pallas kernel here")


# speedup vs baseline: 3.0239x; 3.0239x over previous
import functools
import math

import jax
import jax.numpy as jnp
import numpy as np
from jax import lax
from jax.experimental import pallas as pl
from jax.experimental.pallas import tpu as pltpu
from jax.experimental.pallas import tpu_sc as plsc

N = 10000
E = 320000
F = 128
NE = 10
NB = 8
RMAX = 5.0
P = 5
AVG = 32.0

NC = 2
NS = 16
NW = NC * NS
CH = 128
EPW = 10240
E_PAD = NW * EPW
NCHUNK = EPW // CH
NPAD = 10112
RPT = NPAD // NS
EB = 2048

_f32 = jnp.float32
_i32 = jnp.int32


def _mesh():
    return plsc.VectorSubcoreMesh(
        core_axis_name="c", subcore_axis_name="s", num_cores=NC, num_subcores=NS
    )


def _node_prep_body(attrs_ref, mag_ref, we_ref, wm1_ref, wm2_ref,
                    ae_ref, h0_ref, mhat_ref, m1_ref, m2_ref, e0_ref):
    attrs = attrs_ref[...]
    mag = mag_ref[...]
    h0_ref[...] = jnp.dot(attrs, we_ref[...], preferred_element_type=_f32)
    mn = jnp.sqrt(jnp.sum(mag * mag, axis=1, keepdims=True))
    mhat_ref[...] = mag / (mn + 1e-9)
    x = jnp.tanh(mn)
    wm1 = wm1_ref[...]
    wm2 = wm2_ref[...]
    t_prev = jnp.ones_like(x)
    t_cur = x
    m1 = t_cur * wm1[0:1, :]
    m2 = t_cur * wm2[0:1, :]
    for k in range(1, NB):
        t_next = 2.0 * x * t_cur - t_prev
        t_prev, t_cur = t_cur, t_next
        m1 = m1 + t_cur * wm1[k:k + 1, :]
        m2 = m2 + t_cur * wm2[k:k + 1, :]
    m1_ref[...] = m1
    m2_ref[...] = m2
    e0_ref[...] = jnp.sum(attrs * ae_ref[...], axis=1, keepdims=True)


def _node_prep(node_attrs, magmoms, w_embed, w_m1, w_m2, ae_row):
    return pl.pallas_call(
        _node_prep_body,
        out_shape=(
            jax.ShapeDtypeStruct((N, F), _f32),
            jax.ShapeDtypeStruct((N, 3), _f32),
            jax.ShapeDtypeStruct((N, F), _f32),
            jax.ShapeDtypeStruct((N, F), _f32),
            jax.ShapeDtypeStruct((N, 1), _f32),
        ),
    )(node_attrs, magmoms, w_embed, w_m1, w_m2, ae_row)


def _edge_basis_body(r2_ref, md_ref, wr1_ref, wr2_ref, s1_ref, s2_ref):
    r2 = r2_ref[...]
    md = md_ref[...]
    r = jnp.sqrt(r2)
    rinv = 1.0 / (r + 1e-9)
    mod = 1.0 + md * rinv
    u = r * (1.0 / RMAX)
    uc = jnp.minimum(u, 1.0)
    u2 = u * u
    u4 = u2 * u2
    u5 = u4 * u
    env = 1.0 - 21.0 * u5 + 35.0 * u5 * u - 15.0 * u5 * u2
    env = jnp.where(u < 1.0, env, 0.0)
    coef = math.sqrt(2.0 / RMAX) * rinv * env * mod
    efs = [jnp.sin(uc * (np.pi * n)) * coef for n in range(1, NB + 1)]
    ef_t = jnp.concatenate(efs, axis=0)
    dn = (((0,), (0,)), ((), ()))
    s1_ref[...] = lax.dot_general(ef_t, wr1_ref[...], dn,
                                  preferred_element_type=_f32)
    s2_ref[...] = lax.dot_general(ef_t, wr2_ref[...], dn,
                                  preferred_element_type=_f32)


def _edge_basis(r2, md, w_r1, w_r2):
    grid = E_PAD // EB
    return pl.pallas_call(
        _edge_basis_body,
        grid=(grid,),
        in_specs=[
            pl.BlockSpec((1, EB), lambda i: (0, i)),
            pl.BlockSpec((1, EB), lambda i: (0, i)),
            pl.BlockSpec((NB, F), lambda i: (0, 0)),
            pl.BlockSpec((NB, F), lambda i: (0, 0)),
        ],
        out_specs=(
            pl.BlockSpec((EB, F), lambda i: (i, 0)),
            pl.BlockSpec((EB, F), lambda i: (i, 0)),
        ),
        out_shape=(
            jax.ShapeDtypeStruct((E_PAD, F), _f32),
            jax.ShapeDtypeStruct((E_PAD, F), _f32),
        ),
    )(r2.reshape(1, E_PAD), md.reshape(1, E_PAD), w_r1, w_r2)


def _update_body(h_ref, part_ref, m_ref, out_ref):
    agg = (part_ref[0, 0:N, :] + part_ref[1, 0:N, :]) * (1.0 / AVG)
    out_ref[...] = h_ref[...] + agg * m_ref[...]


def _update(h, part, m):
    return pl.pallas_call(
        _update_body,
        out_shape=jax.ShapeDtypeStruct((N, F), _f32),
    )(h, part, m)


def _final_body(h_ref, part_ref, m_ref, wr_ref, e0_ref, out_ref):
    agg = (part_ref[0, 0:N, :] + part_ref[1, 0:N, :]) * (1.0 / AVG)
    h2 = h_ref[...] + agg * m_ref[...]
    out_ref[...] = jnp.sum(h2 * wr_ref[...], axis=1, keepdims=True) + e0_ref[...]


def _final(h, part, m, wr_row, e0):
    return pl.pallas_call(
        _final_body,
        out_shape=jax.ShapeDtypeStruct((N, 1), _f32),
    )(h, part, m, wr_row, e0)


def _geom_gather(px, py, pz, mx, my, mz, snd, rcv):
    @functools.partial(
        pl.kernel,
        out_type=(
            jax.ShapeDtypeStruct((E_PAD,), _f32),
            jax.ShapeDtypeStruct((E_PAD,), _f32),
        ),
        mesh=_mesh(),
        scratch_types=[
            [pltpu.VMEM((N,), _f32) for _ in range(6)],
            pltpu.VMEM((CH,), _i32),
            pltpu.VMEM((CH,), _i32),
            pltpu.VMEM((CH,), _f32),
            pltpu.VMEM((CH,), _f32),
        ],
        compiler_params=pltpu.CompilerParams(needs_layout_passes=False),
    )
    def body(px_h, py_h, pz_h, mx_h, my_h, mz_h, snd_hbm, rcv_hbm,
             r2_hbm, md_hbm, tabs, idx_s, idx_r, r2_v, md_v):
        for src, dst in zip((px_h, py_h, pz_h, mx_h, my_h, mz_h), tabs):
            pltpu.sync_copy(src, dst)
        tpx, tpy, tpz, tmx, tmy, tmz = tabs
        wid = lax.axis_index("c") * NS + lax.axis_index("s")
        base_w = wid * EPW

        @pl.loop(0, NCHUNK)
        def _chunks(i):
            base = base_w + i * CH
            pltpu.sync_copy(snd_hbm.at[pl.ds(base, CH)], idx_s)
            pltpu.sync_copy(rcv_hbm.at[pl.ds(base, CH)], idx_r)

            @pl.loop(0, CH // 16)
            def _grp(k):
                sl = pl.ds(k * 16, 16)
                i_s = idx_s[sl]
                i_r = idx_r[sl]
                dx = plsc.load_gather(tpx, [i_r]) - plsc.load_gather(tpx, [i_s])
                dy = plsc.load_gather(tpy, [i_r]) - plsc.load_gather(tpy, [i_s])
                dz = plsc.load_gather(tpz, [i_r]) - plsc.load_gather(tpz, [i_s])
                hx = plsc.load_gather(tmx, [i_s])
                hy = plsc.load_gather(tmy, [i_s])
                hz = plsc.load_gather(tmz, [i_s])
                r2_v[sl] = dx * dx + dy * dy + dz * dz
                md_v[sl] = hx * dx + hy * dy + hz * dz

            pltpu.sync_copy(r2_v, r2_hbm.at[pl.ds(base, CH)])
            pltpu.sync_copy(md_v, md_hbm.at[pl.ds(base, CH)])

    return body(px, py, pz, mx, my, mz, snd, rcv)


def _layer(h, scal, snd, rcv):
    @functools.partial(
        pl.kernel,
        out_type=jax.ShapeDtypeStruct((NC, NPAD, F), _f32),
        mesh=_mesh(),
        scratch_types=[
            pltpu.VMEM((CH,), _i32),
            pltpu.VMEM((CH,), _i32),
            pltpu.VMEM((CH, F), _f32),
            pltpu.VMEM((CH, F), _f32),
            pltpu.VMEM((CH, F), _f32),
            pltpu.VMEM_SHARED((NPAD, F), _f32),
            pltpu.SemaphoreType.DMA,
        ],
    )
    def body(h_hbm, scal_hbm, snd_hbm, rcv_hbm, out_hbm,
             idx_s, idx_r, rows, sv, zbuf, agg, sem):
        cid = lax.axis_index("c")
        sid = lax.axis_index("s")
        wid = cid * NS + sid
        z16 = jnp.zeros((16,), _f32)

        @pl.loop(0, CH)
        def _zrow(k):
            for j in range(F // 16):
                zbuf[k, pl.ds(j * 16, 16)] = z16

        off = 0
        for sz in (CH, CH, CH, CH, RPT - 4 * CH):
            pltpu.sync_copy(zbuf.at[pl.ds(0, sz), :],
                            agg.at[pl.ds(sid * RPT + off, sz), :])
            off += sz

        plsc.subcore_barrier()

        base_w = wid * EPW

        @pl.loop(0, NCHUNK)
        def _chunks(i):
            base = base_w + i * CH
            pltpu.sync_copy(snd_hbm.at[pl.ds(base, CH)], idx_s)
            pltpu.sync_copy(rcv_hbm.at[pl.ds(base, CH)], idx_r)
            cg = pltpu.async_copy(h_hbm.at[idx_s], rows, sem)
            pltpu.sync_copy(scal_hbm.at[pl.ds(base, CH), :], sv)
            cg.wait()

            @pl.loop(0, CH)
            def _mul(k):
                for j in range(F // 16):
                    sl = pl.ds(j * 16, 16)
                    rows[k, sl] = rows[k, sl] * sv[k, sl]

            pltpu.sync_copy(rows, agg.at[idx_r], add=True)

        plsc.subcore_barrier()

        pltpu.sync_copy(agg.at[pl.ds(sid * RPT, RPT), :],
                        out_hbm.at[cid, pl.ds(sid * RPT, RPT), :])

    return body(h, scal, snd, rcv)


def kernel(node_attrs, positions, magmoms, edge_index, W_embed, W_r1, W_m1,
           W_r2, W_m2, w_read, atomic_energies):
    pad = jnp.zeros((E_PAD - E,), _i32)
    snd = jnp.concatenate([edge_index[0].astype(_i32), pad])
    rcv = jnp.concatenate([edge_index[1].astype(_i32), pad])
    ae_row = atomic_energies.reshape(1, NE)
    wr_row = w_read.reshape(1, F)

    h0, mhat, m1, m2, e0 = _node_prep(
        node_attrs, magmoms, W_embed, W_m1, W_m2, ae_row)
    px, py, pz = positions[:, 0], positions[:, 1], positions[:, 2]
    mx, my, mz = mhat[:, 0], mhat[:, 1], mhat[:, 2]
    r2, md = _geom_gather(px, py, pz, mx, my, mz, snd, rcv)
    scal1, scal2 = _edge_basis(r2, md, W_r1, W_r2)
    part1 = _layer(h0, scal1, snd, rcv)
    h1 = _update(h0, part1, m1)
    part2 = _layer(h1, scal2, snd, rcv)
    return _final(h1, part2, m2, wr_row, e0)


# speedup vs baseline: 4.1748x; 1.3806x over previous
import functools
import math

import jax
import jax.numpy as jnp
import numpy as np
from jax import lax
from jax.experimental import pallas as pl
from jax.experimental.pallas import tpu as pltpu
from jax.experimental.pallas import tpu_sc as plsc

N = 10000
E = 320000
F = 128
NE = 10
NB = 8
RMAX = 5.0
P = 5
AVG = 32.0

NC = 2
NS = 16
NW = NC * NS
CH = 64
EPW = 10240
E_PAD = NW * EPW
NCHUNK = EPW // CH
NPAD = 10112
RPT = NPAD // NS
EB = 2048

_f32 = jnp.float32
_i32 = jnp.int32


def _mesh():
    return plsc.VectorSubcoreMesh(
        core_axis_name="c", subcore_axis_name="s", num_cores=NC, num_subcores=NS
    )


def _node_prep_body(attrs_ref, mag_ref, we_ref, wm1_ref, wm2_ref,
                    ae_ref, h0_ref, mhat_ref, m1_ref, m2_ref, e0_ref):
    attrs = attrs_ref[...]
    mag = mag_ref[...]
    h0_ref[...] = jnp.dot(attrs, we_ref[...], preferred_element_type=_f32)
    mn = jnp.sqrt(jnp.sum(mag * mag, axis=1, keepdims=True))
    mhat_ref[...] = mag / (mn + 1e-9)
    x = jnp.tanh(mn)
    wm1 = wm1_ref[...]
    wm2 = wm2_ref[...]
    t_prev = jnp.ones_like(x)
    t_cur = x
    m1 = t_cur * wm1[0:1, :]
    m2 = t_cur * wm2[0:1, :]
    for k in range(1, NB):
        t_next = 2.0 * x * t_cur - t_prev
        t_prev, t_cur = t_cur, t_next
        m1 = m1 + t_cur * wm1[k:k + 1, :]
        m2 = m2 + t_cur * wm2[k:k + 1, :]
    m1_ref[...] = m1
    m2_ref[...] = m2
    e0_ref[...] = jnp.sum(attrs * ae_ref[...], axis=1, keepdims=True)


def _node_prep(node_attrs, magmoms, w_embed, w_m1, w_m2, ae_row):
    return pl.pallas_call(
        _node_prep_body,
        out_shape=(
            jax.ShapeDtypeStruct((N, F), _f32),
            jax.ShapeDtypeStruct((N, 3), _f32),
            jax.ShapeDtypeStruct((N, F), _f32),
            jax.ShapeDtypeStruct((N, F), _f32),
            jax.ShapeDtypeStruct((N, 1), _f32),
        ),
    )(node_attrs, magmoms, w_embed, w_m1, w_m2, ae_row)


def _edge_basis_body(r2_ref, md_ref, wr1_ref, wr2_ref, s1_ref, s2_ref):
    r2 = r2_ref[...]
    md = md_ref[...]
    r = jnp.sqrt(r2)
    rinv = 1.0 / (r + 1e-9)
    mod = 1.0 + md * rinv
    u = r * (1.0 / RMAX)
    uc = jnp.minimum(u, 1.0)
    u2 = u * u
    u4 = u2 * u2
    u5 = u4 * u
    env = 1.0 - 21.0 * u5 + 35.0 * u5 * u - 15.0 * u5 * u2
    env = jnp.where(u < 1.0, env, 0.0)
    coef = math.sqrt(2.0 / RMAX) * rinv * env * mod
    efs = [jnp.sin(uc * (np.pi * n)) * coef for n in range(1, NB + 1)]
    ef_t = jnp.concatenate(efs, axis=0)
    dn = (((0,), (0,)), ((), ()))
    s1_ref[...] = lax.dot_general(ef_t, wr1_ref[...], dn,
                                  preferred_element_type=_f32)
    s2_ref[...] = lax.dot_general(ef_t, wr2_ref[...], dn,
                                  preferred_element_type=_f32)


def _edge_basis(r2, md, w_r1, w_r2):
    grid = E_PAD // EB
    return pl.pallas_call(
        _edge_basis_body,
        grid=(grid,),
        in_specs=[
            pl.BlockSpec((1, EB), lambda i: (0, i)),
            pl.BlockSpec((1, EB), lambda i: (0, i)),
            pl.BlockSpec((NB, F), lambda i: (0, 0)),
            pl.BlockSpec((NB, F), lambda i: (0, 0)),
        ],
        out_specs=(
            pl.BlockSpec((EB, F), lambda i: (i, 0)),
            pl.BlockSpec((EB, F), lambda i: (i, 0)),
        ),
        out_shape=(
            jax.ShapeDtypeStruct((E_PAD, F), _f32),
            jax.ShapeDtypeStruct((E_PAD, F), _f32),
        ),
    )(r2.reshape(1, E_PAD), md.reshape(1, E_PAD), w_r1, w_r2)


def _update_body(h_ref, part_ref, m_ref, out_ref):
    agg = (part_ref[0, 0:N, :] + part_ref[1, 0:N, :]) * (1.0 / AVG)
    out_ref[...] = h_ref[...] + agg * m_ref[...]


def _update(h, part, m):
    return pl.pallas_call(
        _update_body,
        out_shape=jax.ShapeDtypeStruct((N, F), _f32),
    )(h, part, m)


def _final_body(h_ref, part_ref, m_ref, wr_ref, e0_ref, out_ref):
    agg = (part_ref[0, 0:N, :] + part_ref[1, 0:N, :]) * (1.0 / AVG)
    h2 = h_ref[...] + agg * m_ref[...]
    out_ref[...] = jnp.sum(h2 * wr_ref[...], axis=1, keepdims=True) + e0_ref[...]


def _final(h, part, m, wr_row, e0):
    return pl.pallas_call(
        _final_body,
        out_shape=jax.ShapeDtypeStruct((N, 1), _f32),
    )(h, part, m, wr_row, e0)


def _geom_gather(px, py, pz, mx, my, mz, snd3, rcv3):
    @functools.partial(
        pl.kernel,
        out_type=(
            jax.ShapeDtypeStruct((E_PAD,), _f32),
            jax.ShapeDtypeStruct((E_PAD,), _f32),
        ),
        mesh=_mesh(),
        scratch_types=[
            [pltpu.VMEM((N,), _f32) for _ in range(6)],
            pltpu.VMEM((EPW // 128, 128), _i32),
            pltpu.VMEM((EPW // 128, 128), _i32),
            pltpu.VMEM((EPW,), _f32),
            pltpu.VMEM((EPW,), _f32),
        ],
        compiler_params=pltpu.CompilerParams(needs_layout_passes=False),
    )
    def body(px_h, py_h, pz_h, mx_h, my_h, mz_h, snd_hbm, rcv_hbm,
             r2_hbm, md_hbm, tabs, snd_all, rcv_all, r2_v, md_v):
        for src, dst in zip((px_h, py_h, pz_h, mx_h, my_h, mz_h), tabs):
            pltpu.sync_copy(src, dst)
        tpx, tpy, tpz, tmx, tmy, tmz = tabs
        wid = lax.axis_index("c") * NS + lax.axis_index("s")
        base_w = wid * EPW
        pltpu.sync_copy(snd_hbm.at[wid], snd_all)
        pltpu.sync_copy(rcv_hbm.at[wid], rcv_all)

        @pl.loop(0, EPW // 128)
        def _chunks(ci):
            @pl.loop(0, 8)
            def _grp(k):
                sl = pl.ds(k * 16, 16)
                i_s = snd_all[ci, sl]
                i_r = rcv_all[ci, sl]
                dx = plsc.load_gather(tpx, [i_r]) - plsc.load_gather(tpx, [i_s])
                dy = plsc.load_gather(tpy, [i_r]) - plsc.load_gather(tpy, [i_s])
                dz = plsc.load_gather(tpz, [i_r]) - plsc.load_gather(tpz, [i_s])
                hx = plsc.load_gather(tmx, [i_s])
                hy = plsc.load_gather(tmy, [i_s])
                hz = plsc.load_gather(tmz, [i_s])
                osl = pl.ds(ci * 128 + k * 16, 16)
                r2_v[osl] = dx * dx + dy * dy + dz * dz
                md_v[osl] = hx * dx + hy * dy + hz * dz

        pltpu.sync_copy(r2_v, r2_hbm.at[pl.ds(base_w, EPW)])
        pltpu.sync_copy(md_v, md_hbm.at[pl.ds(base_w, EPW)])

    return body(px, py, pz, mx, my, mz, snd3, rcv3)


def _layer(h, scal, packed3):
    @functools.partial(
        pl.kernel,
        out_type=jax.ShapeDtypeStruct((NC, NPAD, F), _f32),
        mesh=_mesh(),
        scratch_types=[
            pltpu.VMEM((EPW // 128, 128), _i32),
            pltpu.VMEM((CH,), _i32),
            pltpu.VMEM((CH,), _i32),
            pltpu.VMEM((CH,), _i32),
            pltpu.VMEM((CH,), _i32),
            pltpu.VMEM((CH, F), _f32),
            pltpu.VMEM((CH, F), _f32),
            pltpu.VMEM((CH, F), _f32),
            pltpu.VMEM((CH, F), _f32),
            pltpu.VMEM_SHARED((NPAD, F), _f32),
            pltpu.SemaphoreType.DMA,
            pltpu.SemaphoreType.DMA,
        ],
    )
    def body(h_hbm, scal_hbm, pk_hbm, out_hbm,
             pk_all, is0, is1, ir0, ir1, rows0, rows1, sv0, sv1,
             agg, sem0, sem1):
        cid = lax.axis_index("c")
        sid = lax.axis_index("s")
        wid = cid * NS + sid
        z16 = jnp.zeros((16,), _f32)
        bufs = ((rows0, sv0, is0, ir0, sem0), (rows1, sv1, is1, ir1, sem1))

        @pl.loop(0, CH)
        def _zrow(k):
            for j in range(F // 16):
                rows0[k, pl.ds(j * 16, 16)] = z16

        @pl.loop(0, RPT // CH)
        def _zagg(t):
            pltpu.sync_copy(rows0, agg.at[pl.ds(sid * RPT + t * CH, CH), :])

        pltpu.sync_copy(
            rows0.at[pl.ds(0, RPT - (RPT // CH) * CH), :],
            agg.at[pl.ds(sid * RPT + (RPT // CH) * CH,
                         RPT - (RPT // CH) * CH), :])

        pltpu.sync_copy(pk_hbm.at[wid], pk_all)
        plsc.subcore_barrier()

        base_w = wid * EPW

        def load(slot, row, col, ci):
            rows, sv, isb, irb, sem = bufs[slot]
            for g in range(CH // 16):
                sl = pl.ds(g * 16, 16)
                pk = pk_all[row, pl.ds(col + g * 16, 16)]
                isb[sl] = lax.bitwise_and(pk, 16383)
                irb[sl] = lax.shift_right_logical(pk, 14)
            pltpu.async_copy(h_hbm.at[isb], rows, sem)
            pltpu.async_copy(scal_hbm.at[pl.ds(base_w + ci * CH, CH), :],
                             sv, sem)

        def compute(slot, ci):
            rows, sv, isb, irb, sem = bufs[slot]
            pltpu.make_async_copy(h_hbm.at[isb], rows, sem).wait()
            pltpu.make_async_copy(
                scal_hbm.at[pl.ds(base_w + ci * CH, CH), :], sv, sem).wait()

            @pl.loop(0, CH)
            def _mul(k):
                for j in range(F // 16):
                    sl = pl.ds(j * 16, 16)
                    rows[k, sl] = rows[k, sl] * sv[k, sl]

            pltpu.sync_copy(rows, agg.at[irb], add=True)

        load(0, 0, 0, 0)

        @pl.loop(0, NCHUNK // 2)
        def _pairs(p):
            a = 2 * p
            load(1, p, CH, a + 1)
            compute(0, a)

            @pl.when(p < NCHUNK // 2 - 1)
            def _pre():
                load(0, p + 1, 0, a + 2)

            compute(1, a + 1)

        plsc.subcore_barrier()

        pltpu.sync_copy(agg.at[pl.ds(sid * RPT, RPT), :],
                        out_hbm.at[cid, pl.ds(sid * RPT, RPT), :])

    return body(h, scal, packed3)


def kernel(node_attrs, positions, magmoms, edge_index, W_embed, W_r1, W_m1,
           W_r2, W_m2, w_read, atomic_energies):
    pad = jnp.zeros((E_PAD - E,), _i32)
    snd_p = jnp.concatenate([edge_index[0].astype(_i32), pad])
    rcv_p = jnp.concatenate([edge_index[1].astype(_i32), pad])
    snd3 = snd_p.reshape(NW, EPW // 128, 128)
    rcv3 = rcv_p.reshape(NW, EPW // 128, 128)
    packed3 = jnp.bitwise_or(jnp.left_shift(rcv_p, 14), snd_p).reshape(
        NW, EPW // 128, 128)
    ae_row = atomic_energies.reshape(1, NE)
    wr_row = w_read.reshape(1, F)

    h0, mhat, m1, m2, e0 = _node_prep(
        node_attrs, magmoms, W_embed, W_m1, W_m2, ae_row)
    px, py, pz = positions[:, 0], positions[:, 1], positions[:, 2]
    mx, my, mz = mhat[:, 0], mhat[:, 1], mhat[:, 2]
    r2, md = _geom_gather(px, py, pz, mx, my, mz, snd3, rcv3)
    scal1, scal2 = _edge_basis(r2, md, W_r1, W_r2)
    part1 = _layer(h0, scal1, packed3)
    h1 = _update(h0, part1, m1)
    part2 = _layer(h1, scal2, packed3)
    return _final(h1, part2, m2, wr_row, e0)


# speedup vs baseline: 4.5116x; 1.0807x over previous
import functools
import math

import jax
import jax.numpy as jnp
import numpy as np
from jax import lax
from jax.experimental import pallas as pl
from jax.experimental.pallas import tpu as pltpu
from jax.experimental.pallas import tpu_sc as plsc

N = 10000
E = 320000
F = 128
NE = 10
NB = 8
RMAX = 5.0
P = 5
AVG = 32.0

NC = 2
NS = 16
NW = NC * NS
CH = 64
EPW = 10240
E_PAD = NW * EPW
NCHUNK = EPW // CH
NCH0 = 220
NCH1 = 100
NPAD = 10112
RPT = NPAD // NS
EB = 2048

_f32 = jnp.float32
_i32 = jnp.int32


def _mesh():
    return plsc.VectorSubcoreMesh(
        core_axis_name="c", subcore_axis_name="s", num_cores=NC, num_subcores=NS
    )


def _node_prep_body(attrs_ref, mag_ref, we_ref, wm1_ref, wm2_ref,
                    ae_ref, h0_ref, mhat_ref, m1_ref, m2_ref, e0_ref):
    attrs = attrs_ref[...]
    mag = mag_ref[...]
    h0_ref[...] = jnp.dot(attrs, we_ref[...], preferred_element_type=_f32)
    mn = jnp.sqrt(jnp.sum(mag * mag, axis=1, keepdims=True))
    mhat_ref[...] = mag / (mn + 1e-9)
    x = jnp.tanh(mn)
    wm1 = wm1_ref[...]
    wm2 = wm2_ref[...]
    t_prev = jnp.ones_like(x)
    t_cur = x
    m1 = t_cur * wm1[0:1, :]
    m2 = t_cur * wm2[0:1, :]
    for k in range(1, NB):
        t_next = 2.0 * x * t_cur - t_prev
        t_prev, t_cur = t_cur, t_next
        m1 = m1 + t_cur * wm1[k:k + 1, :]
        m2 = m2 + t_cur * wm2[k:k + 1, :]
    m1_ref[...] = m1
    m2_ref[...] = m2
    e0_ref[...] = jnp.sum(attrs * ae_ref[...], axis=1, keepdims=True)


def _node_prep(node_attrs, magmoms, w_embed, w_m1, w_m2, ae_row):
    return pl.pallas_call(
        _node_prep_body,
        out_shape=(
            jax.ShapeDtypeStruct((N, F), _f32),
            jax.ShapeDtypeStruct((N, 3), _f32),
            jax.ShapeDtypeStruct((N, F), _f32),
            jax.ShapeDtypeStruct((N, F), _f32),
            jax.ShapeDtypeStruct((N, 1), _f32),
        ),
    )(node_attrs, magmoms, w_embed, w_m1, w_m2, ae_row)


def _edge_basis_body(r2_ref, md_ref, wr1_ref, wr2_ref, s1_ref, s2_ref):
    r2 = r2_ref[...]
    md = md_ref[...]
    r = jnp.sqrt(r2)
    rinv = 1.0 / (r + 1e-9)
    mod = 1.0 + md * rinv
    u = r * (1.0 / RMAX)
    uc = jnp.minimum(u, 1.0)
    u2 = u * u
    u4 = u2 * u2
    u5 = u4 * u
    env = 1.0 - 21.0 * u5 + 35.0 * u5 * u - 15.0 * u5 * u2
    env = jnp.where(u < 1.0, env, 0.0)
    coef = math.sqrt(2.0 / RMAX) * rinv * env * mod
    efs = [jnp.sin(uc * (np.pi * n)) * coef for n in range(1, NB + 1)]
    ef_t = jnp.concatenate(efs, axis=0)
    dn = (((0,), (0,)), ((), ()))
    s1_ref[...] = lax.dot_general(ef_t, wr1_ref[...], dn,
                                  preferred_element_type=_f32)
    s2_ref[...] = lax.dot_general(ef_t, wr2_ref[...], dn,
                                  preferred_element_type=_f32)


def _edge_basis(r2, md, w_r1, w_r2):
    grid = E_PAD // EB
    return pl.pallas_call(
        _edge_basis_body,
        grid=(grid,),
        in_specs=[
            pl.BlockSpec((1, EB), lambda i: (0, i)),
            pl.BlockSpec((1, EB), lambda i: (0, i)),
            pl.BlockSpec((NB, F), lambda i: (0, 0)),
            pl.BlockSpec((NB, F), lambda i: (0, 0)),
        ],
        out_specs=(
            pl.BlockSpec((EB, F), lambda i: (i, 0)),
            pl.BlockSpec((EB, F), lambda i: (i, 0)),
        ),
        out_shape=(
            jax.ShapeDtypeStruct((E_PAD, F), _f32),
            jax.ShapeDtypeStruct((E_PAD, F), _f32),
        ),
    )(r2.reshape(1, E_PAD), md.reshape(1, E_PAD), w_r1, w_r2)


def _update_body(h_ref, part_ref, m_ref, out_ref):
    agg = (part_ref[0, 0:N, :] + part_ref[1, 0:N, :]) * (1.0 / AVG)
    out_ref[...] = h_ref[...] + agg * m_ref[...]


def _update(h, part, m):
    return pl.pallas_call(
        _update_body,
        out_shape=jax.ShapeDtypeStruct((N, F), _f32),
    )(h, part, m)


def _final_body(h_ref, part_ref, m_ref, wr_ref, e0_ref, out_ref):
    agg = (part_ref[0, 0:N, :] + part_ref[1, 0:N, :]) * (1.0 / AVG)
    h2 = h_ref[...] + agg * m_ref[...]
    out_ref[...] = jnp.sum(h2 * wr_ref[...], axis=1, keepdims=True) + e0_ref[...]


def _final(h, part, m, wr_row, e0):
    return pl.pallas_call(
        _final_body,
        out_shape=jax.ShapeDtypeStruct((N, 1), _f32),
    )(h, part, m, wr_row, e0)


def _geom_gather(px, py, pz, mx, my, mz, snd3, rcv3):
    @functools.partial(
        pl.kernel,
        out_type=(
            jax.ShapeDtypeStruct((E_PAD,), _f32),
            jax.ShapeDtypeStruct((E_PAD,), _f32),
        ),
        mesh=_mesh(),
        scratch_types=[
            [pltpu.VMEM((N,), _f32) for _ in range(6)],
            pltpu.VMEM((EPW // 128, 128), _i32),
            pltpu.VMEM((EPW // 128, 128), _i32),
            pltpu.VMEM((EPW,), _f32),
            pltpu.VMEM((EPW,), _f32),
        ],
        compiler_params=pltpu.CompilerParams(needs_layout_passes=False),
    )
    def body(px_h, py_h, pz_h, mx_h, my_h, mz_h, snd_hbm, rcv_hbm,
             r2_hbm, md_hbm, tabs, snd_all, rcv_all, r2_v, md_v):
        for src, dst in zip((px_h, py_h, pz_h, mx_h, my_h, mz_h), tabs):
            pltpu.sync_copy(src, dst)
        tpx, tpy, tpz, tmx, tmy, tmz = tabs
        wid = lax.axis_index("c") * NS + lax.axis_index("s")
        base_w = wid * EPW
        pltpu.sync_copy(snd_hbm.at[wid], snd_all)
        pltpu.sync_copy(rcv_hbm.at[wid], rcv_all)

        @pl.loop(0, EPW // 128)
        def _chunks(ci):
            @pl.loop(0, 8)
            def _grp(k):
                sl = pl.ds(k * 16, 16)
                i_s = snd_all[ci, sl]
                i_r = rcv_all[ci, sl]
                dx = plsc.load_gather(tpx, [i_r]) - plsc.load_gather(tpx, [i_s])
                dy = plsc.load_gather(tpy, [i_r]) - plsc.load_gather(tpy, [i_s])
                dz = plsc.load_gather(tpz, [i_r]) - plsc.load_gather(tpz, [i_s])
                hx = plsc.load_gather(tmx, [i_s])
                hy = plsc.load_gather(tmy, [i_s])
                hz = plsc.load_gather(tmz, [i_s])
                osl = pl.ds(ci * 128 + k * 16, 16)
                r2_v[osl] = dx * dx + dy * dy + dz * dz
                md_v[osl] = hx * dx + hy * dy + hz * dz

        pltpu.sync_copy(r2_v, r2_hbm.at[pl.ds(base_w, EPW)])
        pltpu.sync_copy(md_v, md_hbm.at[pl.ds(base_w, EPW)])

    return body(px, py, pz, mx, my, mz, snd3, rcv3)


def _layer(h, scal, packed3):
    @functools.partial(
        pl.kernel,
        out_type=jax.ShapeDtypeStruct((NC, NPAD, F), _f32),
        mesh=_mesh(),
        scratch_types=[
            pltpu.VMEM((NCH0 * CH,), _i32),
            pltpu.VMEM((CH,), _i32),
            pltpu.VMEM((CH,), _i32),
            pltpu.VMEM((CH,), _i32),
            pltpu.VMEM((CH,), _i32),
            pltpu.VMEM((CH, F), _f32),
            pltpu.VMEM((CH, F), _f32),
            pltpu.VMEM((CH, F), _f32),
            pltpu.VMEM((CH, F), _f32),
            pltpu.VMEM_SHARED((NPAD, F), _f32),
            pltpu.SemaphoreType.DMA,
            pltpu.SemaphoreType.DMA,
        ],
    )
    def body(h_hbm, scal_hbm, pk_hbm, out_hbm,
             pk_all, is0, is1, ir0, ir1, rows0, rows1, sv0, sv1,
             agg, sem0, sem1):
        cid = lax.axis_index("c")
        sid = lax.axis_index("s")
        z16 = jnp.zeros((16,), _f32)
        bufs = ((rows0, sv0, is0, ir0, sem0), (rows1, sv1, is1, ir1, sem1))

        @pl.loop(0, CH)
        def _zrow(k):
            for j in range(F // 16):
                rows0[k, pl.ds(j * 16, 16)] = z16

        @pl.loop(0, RPT // CH)
        def _zagg(t):
            pltpu.sync_copy(rows0, agg.at[pl.ds(sid * RPT + t * CH, CH), :])

        pltpu.sync_copy(
            rows0.at[pl.ds(0, RPT - (RPT // CH) * CH), :],
            agg.at[pl.ds(sid * RPT + (RPT // CH) * CH,
                         RPT - (RPT // CH) * CH), :])

        my_chunks = jnp.where(cid == 0, NCH0, NCH1)
        base_chunk = jnp.where(cid == 0, sid * NCH0, NS * NCH0 + sid * NCH1)

        @pl.when(cid == 0)
        def _pk0():
            pltpu.sync_copy(pk_hbm.at[pl.ds(sid * (NCH0 * CH), NCH0 * CH)],
                            pk_all)

        @pl.when(cid == 1)
        def _pk1():
            pltpu.sync_copy(
                pk_hbm.at[pl.ds(NS * NCH0 * CH + sid * (NCH1 * CH),
                                NCH1 * CH)],
                pk_all.at[pl.ds(0, NCH1 * CH)])

        plsc.subcore_barrier()

        def load(slot, ci):
            rows, sv, isb, irb, sem = bufs[slot]
            for g in range(CH // 16):
                sl = pl.ds(g * 16, 16)
                pk = pk_all[pl.ds(ci * CH + g * 16, 16)]
                isb[sl] = lax.bitwise_and(pk, 16383)
                irb[sl] = lax.shift_right_logical(pk, 14)
            gchunk = base_chunk + ci
            pltpu.async_copy(h_hbm.at[isb], rows, sem)
            pltpu.async_copy(scal_hbm.at[pl.ds(gchunk * CH, CH), :], sv, sem)

        def compute(slot, ci):
            rows, sv, isb, irb, sem = bufs[slot]
            gchunk = base_chunk + ci
            pltpu.make_async_copy(h_hbm.at[isb], rows, sem).wait()
            pltpu.make_async_copy(
                scal_hbm.at[pl.ds(gchunk * CH, CH), :], sv, sem).wait()

            @pl.loop(0, CH)
            def _mul(k):
                for j in range(F // 16):
                    sl = pl.ds(j * 16, 16)
                    rows[k, sl] = rows[k, sl] * sv[k, sl]

            pltpu.sync_copy(rows, agg.at[irb], add=True)

        load(0, 0)
        npairs = jnp.where(cid == 0, NCH0 // 2, NCH1 // 2)

        @pl.loop(0, npairs)
        def _pairs(p):
            a = 2 * p
            load(1, a + 1)
            compute(0, a)

            @pl.when(p < npairs - 1)
            def _pre():
                load(0, a + 2)

            compute(1, a + 1)

        plsc.subcore_barrier()

        pltpu.sync_copy(agg.at[pl.ds(sid * RPT, RPT), :],
                        out_hbm.at[cid, pl.ds(sid * RPT, RPT), :])

    return body(h, scal, packed3)


def kernel(node_attrs, positions, magmoms, edge_index, W_embed, W_r1, W_m1,
           W_r2, W_m2, w_read, atomic_energies):
    pad = jnp.zeros((E_PAD - E,), _i32)
    snd_p = jnp.concatenate([edge_index[0].astype(_i32), pad])
    rcv_p = jnp.concatenate([edge_index[1].astype(_i32), pad])
    snd3 = snd_p.reshape(NW, EPW // 128, 128)
    rcv3 = rcv_p.reshape(NW, EPW // 128, 128)
    packed = jnp.bitwise_or(jnp.left_shift(rcv_p, 14), snd_p)
    ae_row = atomic_energies.reshape(1, NE)
    wr_row = w_read.reshape(1, F)

    h0, mhat, m1, m2, e0 = _node_prep(
        node_attrs, magmoms, W_embed, W_m1, W_m2, ae_row)
    px, py, pz = positions[:, 0], positions[:, 1], positions[:, 2]
    mx, my, mz = mhat[:, 0], mhat[:, 1], mhat[:, 2]
    r2, md = _geom_gather(px, py, pz, mx, my, mz, snd3, rcv3)
    scal1, scal2 = _edge_basis(r2, md, W_r1, W_r2)
    part1 = _layer(h0, scal1, packed)
    h1 = _update(h0, part1, m1)
    part2 = _layer(h1, scal2, packed)
    return _final(h1, part2, m2, wr_row, e0)
